# Initial kernel scaffold; baseline (speedup 1.0000x reference)
import jax
import jax.numpy as jnp
from jax.experimental import pallas as pl


def kernel(rating, target_rating, item, target_item):
    raise NotImplementedError("write your pallas kernel here")


# speedup vs baseline: 29.5830x; 29.5830x over previous
import functools

import jax
import jax.numpy as jnp
from jax import lax
from jax.experimental import pallas as pl
from jax.experimental.pallas import tpu as pltpu
from jax.experimental.pallas import tpu_sc as plsc

NUM_ITEMS = 1_000_000
NIP = 1 << 20
N = 3_276_800
M = 16_384

NROWS = N // 128
NSUB = 16
ROWS_PER_TILE = NROWS // NSUB
CHUNK_ROWS = 32
NCHUNKS = ROWS_PER_TILE // CHUNK_ROWS
CHUNK = CHUNK_ROWS * 128
ACC_TILE = NIP // NSUB
TGT_ROWS_PER_TILE = (M // 128) // NSUB


def _fill_const(ref, n, value):
    vec = jnp.full((16,), value, jnp.float32)

    def body(i, _):
        ref[pl.ds(i * 16, 16)] = vec
        return 0

    lax.fori_loop(0, n // 16, body, 0)


def _sc_accumulate(item2d, rating, tgt2d):
    mesh = plsc.VectorSubcoreMesh(core_axis_name="c", subcore_axis_name="s")

    @functools.partial(
        pl.kernel,
        mesh=mesh,
        out_type=[
            jax.ShapeDtypeStruct((NIP,), jnp.float32),
            jax.ShapeDtypeStruct((NIP,), jnp.float32),
            jax.ShapeDtypeStruct((128, 128), jnp.float32),
            jax.ShapeDtypeStruct((128, 128), jnp.float32),
        ],
        scratch_types=[
            pltpu.VMEM_SHARED((NIP,), jnp.float32),
            pltpu.VMEM((CHUNK_ROWS, 128), jnp.int32),
            pltpu.VMEM((CHUNK,), jnp.float32),
            pltpu.VMEM((TGT_ROWS_PER_TILE, 128), jnp.int32),
            pltpu.VMEM((TGT_ROWS_PER_TILE, 128), jnp.float32),
        ],
    )
    def k(item_ref, rating_ref, tgt_ref, base_ref, count_ref, bt_ref, ct_ref,
          acc, idx_v, val_v, tidx_v, tout_v):
        cid = lax.axis_index("c")
        sid = lax.axis_index("s")

        _fill_const(val_v, CHUNK, 0.0)

        def zbody(kk, _):
            pltpu.sync_copy(val_v, acc.at[pl.ds(sid * ACC_TILE + kk * CHUNK, CHUNK)])
            return 0

        lax.fori_loop(0, ACC_TILE // CHUNK, zbody, 0)
        plsc.subcore_barrier()

        @pl.when(cid == 1)
        def _():
            _fill_const(val_v, CHUNK, 1.0)

        row0 = sid * ROWS_PER_TILE

        def chunk_body(c, _):
            row = row0 + c * CHUNK_ROWS
            pltpu.sync_copy(item_ref.at[pl.ds(row, CHUNK_ROWS)], idx_v)

            @pl.when(cid == 0)
            def _():
                pltpu.sync_copy(rating_ref.at[pl.ds(row * 128, CHUNK)], val_v)

            for j in range(CHUNK_ROWS):
                pltpu.sync_copy(val_v.at[pl.ds(j * 128, 128)],
                                acc.at[idx_v.at[j]], add=True)
            return 0

        lax.fori_loop(0, NCHUNKS, chunk_body, 0)
        plsc.subcore_barrier()

        @pl.when(cid == 0)
        def _():
            pltpu.sync_copy(acc.at[pl.ds(sid * ACC_TILE, ACC_TILE)],
                            base_ref.at[pl.ds(sid * ACC_TILE, ACC_TILE)])

        @pl.when(cid == 1)
        def _():
            pltpu.sync_copy(acc.at[pl.ds(sid * ACC_TILE, ACC_TILE)],
                            count_ref.at[pl.ds(sid * ACC_TILE, ACC_TILE)])

        pltpu.sync_copy(tgt_ref.at[pl.ds(sid * TGT_ROWS_PER_TILE, TGT_ROWS_PER_TILE)],
                        tidx_v)
        for j in range(TGT_ROWS_PER_TILE):
            pltpu.sync_copy(acc.at[tidx_v.at[j]], tout_v.at[j])

        @pl.when(cid == 0)
        def _():
            pltpu.sync_copy(tout_v, bt_ref.at[pl.ds(sid * TGT_ROWS_PER_TILE,
                                                    TGT_ROWS_PER_TILE)])

        @pl.when(cid == 1)
        def _():
            pltpu.sync_copy(tout_v, ct_ref.at[pl.ds(sid * TGT_ROWS_PER_TILE,
                                                    TGT_ROWS_PER_TILE)])

    return k(item2d, rating, tgt2d)


_RB = 8


def _tc_finish_body(base_ref, count_ref, bt_ref, ct_ref, tgtr_ref,
                    tr_ref, loss_ref, acc):
    i = pl.program_id(0)

    @pl.when(i == 0)
    def _():
        acc[0] = 0.0
        acc[1] = 0.0

    b = base_ref[...]
    c = count_ref[...]
    nzm = c != 0.0
    ratio = jnp.where(nzm, b / jnp.where(nzm, c, 1.0), 0.0)
    acc[0] += jnp.sum(ratio)
    acc[1] += jnp.sum(nzm.astype(jnp.float32))

    @pl.when(i == _RB - 1)
    def _():
        gmean = acc[0] / jnp.maximum(acc[1], 1.0)
        bt = bt_ref[...]
        ct = ct_ref[...]
        tr = jnp.where(ct == 0.0, gmean, bt / (ct + 1e-10))
        tr_ref[...] = tr
        loss_ref[0, 0] = jnp.mean((tr - tgtr_ref[...]) ** 2)


def _tc_finish(base2d, count2d, bt, ct, tgtr):
    blk = 1024 // _RB
    return pl.pallas_call(
        _tc_finish_body,
        grid=(_RB,),
        in_specs=[
            pl.BlockSpec((blk, 1024), lambda i: (i, 0)),
            pl.BlockSpec((blk, 1024), lambda i: (i, 0)),
            pl.BlockSpec((128, 128), lambda i: (0, 0)),
            pl.BlockSpec((128, 128), lambda i: (0, 0)),
            pl.BlockSpec((128, 128), lambda i: (0, 0)),
        ],
        out_specs=[
            pl.BlockSpec((128, 128), lambda i: (0, 0)),
            pl.BlockSpec((1, 1), lambda i: (0, 0), memory_space=pltpu.SMEM),
        ],
        out_shape=[
            jax.ShapeDtypeStruct((128, 128), jnp.float32),
            jax.ShapeDtypeStruct((1, 1), jnp.float32),
        ],
        scratch_shapes=[pltpu.SMEM((2,), jnp.float32)],
    )(base2d, count2d, bt, ct, tgtr)


def kernel(rating, target_rating, item, target_item):
    item2d = item.astype(jnp.int32).reshape(NROWS, 128)
    tgt2d = target_item.astype(jnp.int32).reshape(128, 128)
    base, count, bt, ct = _sc_accumulate(item2d, rating, tgt2d)
    tr2d, loss = _tc_finish(base.reshape(1024, 1024), count.reshape(1024, 1024),
                            bt, ct, target_rating.reshape(128, 128))
    return tr2d.reshape(M), loss.reshape(())


# speedup vs baseline: 69.9303x; 2.3639x over previous
import functools

import jax
import jax.numpy as jnp
from jax import lax
from jax.experimental import pallas as pl
from jax.experimental.pallas import tpu as pltpu
from jax.experimental.pallas import tpu_sc as plsc

NUM_ITEMS = 1_000_000
NIP = 1 << 20
N = 3_276_800
M = 16_384

NROWS = N // 128
NSUB = 16
ROWS_PER_TILE = NROWS // NSUB
CHUNK_ROWS = 32
NCHUNKS = ROWS_PER_TILE // CHUNK_ROWS
CHUNK = CHUNK_ROWS * 128
ACC_TILE = NIP // NSUB
TGT_ROWS_PER_TILE = (M // 128) // NSUB


def _fill_const(ref, n, value):
    vec = jnp.full((16,), value, jnp.float32)

    def body(i, _):
        ref[pl.ds(i * 16, 16)] = vec
        return 0

    lax.fori_loop(0, n // 16, body, 0)


def _sc_accumulate(item2d, rating, tgt2d):
    mesh = plsc.VectorSubcoreMesh(core_axis_name="c", subcore_axis_name="s")

    @functools.partial(
        pl.kernel,
        mesh=mesh,
        out_type=[
            jax.ShapeDtypeStruct((NIP,), jnp.float32),
            jax.ShapeDtypeStruct((NIP,), jnp.float32),
            jax.ShapeDtypeStruct((128, 128), jnp.float32),
            jax.ShapeDtypeStruct((128, 128), jnp.float32),
        ],
        scratch_types=[
            pltpu.VMEM_SHARED((NIP,), jnp.float32),
            pltpu.VMEM((2, CHUNK_ROWS, 128), jnp.int32),
            pltpu.VMEM((2, CHUNK), jnp.float32),
            pltpu.VMEM((TGT_ROWS_PER_TILE, 128), jnp.int32),
            pltpu.VMEM((TGT_ROWS_PER_TILE, 128), jnp.float32),
            pltpu.SemaphoreType.DMA,
            pltpu.SemaphoreType.DMA,
        ],
    )
    def k(item_ref, rating_ref, tgt_ref, zeros_ref, base_ref, count_ref,
          bt_ref, ct_ref, acc, idx_v, val_v, tidx_v, tout_v, sem_in,
          sem_sc):
        cid = lax.axis_index("c")
        sid = lax.axis_index("s")
        row0 = sid * ROWS_PER_TILE

        def in_copies(c, b):
            row = row0 + c * CHUNK_ROWS
            yield pltpu.make_async_copy(item_ref.at[pl.ds(row, CHUNK_ROWS)],
                                        idx_v.at[b], sem_in)
            yield pltpu.make_async_copy(rating_ref.at[pl.ds(row * 128, CHUNK)],
                                        val_v.at[b], sem_in)

        def start_in(c, b):
            for ii, cp in enumerate(in_copies(c, b)):
                if ii == 0:
                    cp.start()
                else:
                    @pl.when(cid == 0)
                    def _():
                        cp.start()

        def wait_in(c, b):
            for ii, cp in enumerate(in_copies(c, b)):
                if ii == 0:
                    cp.wait()
                else:
                    @pl.when(cid == 0)
                    def _():
                        cp.wait()

        start_in(0, 0)

        pltpu.sync_copy(zeros_ref.at[pl.ds(sid * ACC_TILE, ACC_TILE)],
                        acc.at[pl.ds(sid * ACC_TILE, ACC_TILE)])
        plsc.subcore_barrier()

        @pl.when(cid == 1)
        def _():
            _fill_const(val_v.at[0], CHUNK, 1.0)
            _fill_const(val_v.at[1], CHUNK, 1.0)

        def do_chunk(c, b):
            wait_in(c, b)

            @pl.when(c + 1 < NCHUNKS)
            def _():
                start_in(c + 1, 1 - b)

            copies = [
                pltpu.async_copy(val_v.at[b].at[pl.ds(j * 128, 128)],
                                 acc.at[idx_v.at[b].at[j]], sem_sc, add=True)
                for j in range(CHUNK_ROWS)
            ]
            for cp in copies:
                cp.wait()

        def chunk_body(g, _):
            do_chunk(g * 2, 0)
            do_chunk(g * 2 + 1, 1)
            return 0

        lax.fori_loop(0, NCHUNKS // 2, chunk_body, 0)
        plsc.subcore_barrier()

        @pl.when(cid == 0)
        def _():
            pltpu.sync_copy(acc.at[pl.ds(sid * ACC_TILE, ACC_TILE)],
                            base_ref.at[pl.ds(sid * ACC_TILE, ACC_TILE)])

        @pl.when(cid == 1)
        def _():
            pltpu.sync_copy(acc.at[pl.ds(sid * ACC_TILE, ACC_TILE)],
                            count_ref.at[pl.ds(sid * ACC_TILE, ACC_TILE)])

        pltpu.sync_copy(tgt_ref.at[pl.ds(sid * TGT_ROWS_PER_TILE, TGT_ROWS_PER_TILE)],
                        tidx_v)
        for j in range(TGT_ROWS_PER_TILE):
            pltpu.sync_copy(acc.at[tidx_v.at[j]], tout_v.at[j])

        @pl.when(cid == 0)
        def _():
            pltpu.sync_copy(tout_v, bt_ref.at[pl.ds(sid * TGT_ROWS_PER_TILE,
                                                    TGT_ROWS_PER_TILE)])

        @pl.when(cid == 1)
        def _():
            pltpu.sync_copy(tout_v, ct_ref.at[pl.ds(sid * TGT_ROWS_PER_TILE,
                                                    TGT_ROWS_PER_TILE)])

    return k(item2d, rating, tgt2d, jnp.zeros((NIP,), jnp.float32))


_RB = 8


def _tc_finish_body(base_ref, count_ref, bt_ref, ct_ref, tgtr_ref,
                    tr_ref, loss_ref, acc):
    i = pl.program_id(0)

    @pl.when(i == 0)
    def _():
        acc[0] = 0.0
        acc[1] = 0.0

    b = base_ref[...]
    c = count_ref[...]
    nzm = c != 0.0
    ratio = jnp.where(nzm, b / jnp.where(nzm, c, 1.0), 0.0)
    acc[0] += jnp.sum(ratio)
    acc[1] += jnp.sum(nzm.astype(jnp.float32))

    @pl.when(i == _RB - 1)
    def _():
        gmean = acc[0] / jnp.maximum(acc[1], 1.0)
        bt = bt_ref[...]
        ct = ct_ref[...]
        tr = jnp.where(ct == 0.0, gmean, bt / (ct + 1e-10))
        tr_ref[...] = tr
        loss_ref[0, 0] = jnp.mean((tr - tgtr_ref[...]) ** 2)


def _tc_finish(base2d, count2d, bt, ct, tgtr):
    blk = 1024 // _RB
    return pl.pallas_call(
        _tc_finish_body,
        grid=(_RB,),
        in_specs=[
            pl.BlockSpec((blk, 1024), lambda i: (i, 0)),
            pl.BlockSpec((blk, 1024), lambda i: (i, 0)),
            pl.BlockSpec((128, 128), lambda i: (0, 0)),
            pl.BlockSpec((128, 128), lambda i: (0, 0)),
            pl.BlockSpec((128, 128), lambda i: (0, 0)),
        ],
        out_specs=[
            pl.BlockSpec((128, 128), lambda i: (0, 0)),
            pl.BlockSpec((1, 1), lambda i: (0, 0), memory_space=pltpu.SMEM),
        ],
        out_shape=[
            jax.ShapeDtypeStruct((128, 128), jnp.float32),
            jax.ShapeDtypeStruct((1, 1), jnp.float32),
        ],
        scratch_shapes=[pltpu.SMEM((2,), jnp.float32)],
    )(base2d, count2d, bt, ct, tgtr)


def kernel(rating, target_rating, item, target_item):
    item2d = item.astype(jnp.int32).reshape(NROWS, 128)
    tgt2d = target_item.astype(jnp.int32).reshape(128, 128)
    base, count, bt, ct = _sc_accumulate(item2d, rating, tgt2d)
    tr2d, loss = _tc_finish(base.reshape(1024, 1024), count.reshape(1024, 1024),
                            bt, ct, target_rating.reshape(128, 128))
    return tr2d.reshape(M), loss.reshape(())
